# baseline (speedup 1.0000x reference)
import jax
import jax.numpy as jnp
from jax.experimental import pallas as pl
from jax.experimental.pallas import tpu as pltpu

_HIDDEN = 2048
_NUM_EXPERTS = 16
_HALF = 512
_STEP = 2 * _HALF
_NBUF = 8


def _epilogue(logits, scores_ref, idx_ref, rows):
    m = jnp.max(logits, axis=-1, keepdims=True)
    e = jnp.exp(logits - m)
    scores_ref[rows, :] = e / jnp.sum(e, axis=-1, keepdims=True)

    iota = jax.lax.broadcasted_iota(jnp.int32, logits.shape, 1)
    big = jnp.int32(_NUM_EXPERTS)
    idx0 = jnp.min(jnp.where(logits == m, iota, big), axis=-1, keepdims=True)
    masked = jnp.where(iota == idx0, -jnp.inf, logits)
    m1 = jnp.max(masked, axis=-1, keepdims=True)
    idx1 = jnp.min(jnp.where(masked == m1, iota, big), axis=-1, keepdims=True)

    lane = jax.lax.broadcasted_iota(jnp.int32, (_HALF, 2), 1)
    idx_ref[rows, :] = jnp.where(lane == 0, idx0, idx1)


def _router_kernel(x_hbm, x2_ref, w_ref, scores_ref, idx_ref, buf, wt, sem):
    i = pl.program_id(0)
    n = pl.num_programs(0)

    def _start_chunk(c, slot):
        pltpu.make_async_copy(
            x_hbm.at[pl.ds(c * _STEP, _HALF), :], buf.at[slot], sem.at[slot]
        ).start()

    @pl.when(i == 0)
    def _prologue():
        wt[...] = w_ref[...].T
        for c in range(min(_NBUF, pl.num_programs(0))):
            _start_chunk(c, c)

    @pl.when((i > 0) & (i + _NBUF - 1 < n))
    def _fetch():
        c = i + _NBUF - 1
        _start_chunk(c, jax.lax.rem(c, _NBUF))

    slot = jax.lax.rem(i, _NBUF)
    pltpu.make_async_copy(
        x_hbm.at[pl.ds(i * _STEP, _HALF), :], buf.at[slot], sem.at[slot]
    ).wait()

    w = wt[...]
    dims = (((1,), (0,)), ((), ()))
    logits1 = jax.lax.dot_general(
        buf[slot], w, dims, preferred_element_type=jnp.float32
    )
    _epilogue(logits1, scores_ref, idx_ref, pl.ds(0, _HALF))
    logits2 = jax.lax.dot_general(
        x2_ref[...], w, dims, preferred_element_type=jnp.float32
    )
    _epilogue(logits2, scores_ref, idx_ref, pl.ds(_HALF, _HALF))


def kernel(hidden_states, weight):
    n_tokens = hidden_states.shape[0]
    grid = n_tokens // _STEP
    return pl.pallas_call(
        _router_kernel,
        grid=(grid,),
        in_specs=[
            pl.BlockSpec(memory_space=pl.ANY),
            pl.BlockSpec((_HALF, _HIDDEN), lambda i: (2 * i + 1, 0)),
            pl.BlockSpec((_NUM_EXPERTS, _HIDDEN), lambda i: (0, 0)),
        ],
        out_specs=[
            pl.BlockSpec((_STEP, _NUM_EXPERTS), lambda i: (i, 0)),
            pl.BlockSpec((_STEP, 2), lambda i: (i, 0)),
        ],
        out_shape=[
            jax.ShapeDtypeStruct((n_tokens, _NUM_EXPERTS), jnp.float32),
            jax.ShapeDtypeStruct((n_tokens, 2), jnp.int32),
        ],
        scratch_shapes=[
            pltpu.VMEM((_NBUF, _HALF, _HIDDEN), jnp.float32),
            pltpu.VMEM((_HIDDEN, _NUM_EXPERTS), jnp.float32),
            pltpu.SemaphoreType.DMA((_NBUF,)),
        ],
        compiler_params=pltpu.CompilerParams(
            dimension_semantics=("arbitrary",),
        ),
    )(hidden_states, hidden_states, weight)


# speedup vs baseline: 1.0731x; 1.0731x over previous
import jax
import jax.numpy as jnp
from jax.experimental import pallas as pl
from jax.experimental.pallas import tpu as pltpu

_HIDDEN = 2048
_NUM_EXPERTS = 16
_CHUNK = 512
_NBUF = 8
_NSPLIT = 1
_SUB = _CHUNK // _NSPLIT


def _router_kernel(x_hbm, w_ref, scores_ref, idx_ref, buf, wt, sem):
    i = pl.program_id(0)
    n = pl.num_programs(0)

    def _start_chunk(c, slot):
        for q in range(_NSPLIT):
            pltpu.make_async_copy(
                x_hbm.at[pl.ds(c * _CHUNK + q * _SUB, _SUB), :],
                buf.at[slot, pl.ds(q * _SUB, _SUB), :],
                sem.at[slot, q],
            ).start()

    @pl.when(i == 0)
    def _prologue():
        wt[...] = w_ref[...].T
        for c in range(_NBUF):
            _start_chunk(c, c)

    @pl.when((i > 0) & (i + _NBUF - 1 < n))
    def _fetch():
        c = i + _NBUF - 1
        _start_chunk(c, jax.lax.rem(c, _NBUF))

    slot = jax.lax.rem(i, _NBUF)
    for q in range(_NSPLIT):
        pltpu.make_async_copy(
            x_hbm.at[pl.ds(i * _CHUNK + q * _SUB, _SUB), :],
            buf.at[slot, pl.ds(q * _SUB, _SUB), :],
            sem.at[slot, q],
        ).wait()

    x = buf[slot]
    logits = jax.lax.dot_general(
        x, wt[...], (((1,), (0,)), ((), ())), preferred_element_type=jnp.float32
    )

    m = jnp.max(logits, axis=-1, keepdims=True)
    e = jnp.exp(logits - m)
    scores_ref[...] = e / jnp.sum(e, axis=-1, keepdims=True)

    iota = jax.lax.broadcasted_iota(jnp.int32, logits.shape, 1)
    big = jnp.int32(_NUM_EXPERTS)
    idx0 = jnp.min(jnp.where(logits == m, iota, big), axis=-1, keepdims=True)
    masked = jnp.where(iota == idx0, -jnp.inf, logits)
    m1 = jnp.max(masked, axis=-1, keepdims=True)
    idx1 = jnp.min(jnp.where(masked == m1, iota, big), axis=-1, keepdims=True)

    lane = jax.lax.broadcasted_iota(jnp.int32, (_CHUNK, 2), 1)
    idx_ref[...] = jnp.where(lane == 0, idx0, idx1)


def kernel(hidden_states, weight):
    n_tokens = hidden_states.shape[0]
    grid = n_tokens // _CHUNK
    return pl.pallas_call(
        _router_kernel,
        grid=(grid,),
        in_specs=[
            pl.BlockSpec(memory_space=pl.ANY),
            pl.BlockSpec((_NUM_EXPERTS, _HIDDEN), lambda i: (0, 0)),
        ],
        out_specs=[
            pl.BlockSpec((_CHUNK, _NUM_EXPERTS), lambda i: (i, 0)),
            pl.BlockSpec((_CHUNK, 2), lambda i: (i, 0)),
        ],
        out_shape=[
            jax.ShapeDtypeStruct((n_tokens, _NUM_EXPERTS), jnp.float32),
            jax.ShapeDtypeStruct((n_tokens, 2), jnp.int32),
        ],
        scratch_shapes=[
            pltpu.VMEM((_NBUF, _CHUNK, _HIDDEN), jnp.float32),
            pltpu.VMEM((_HIDDEN, _NUM_EXPERTS), jnp.float32),
            pltpu.SemaphoreType.DMA((_NBUF, _NSPLIT)),
        ],
        compiler_params=pltpu.CompilerParams(
            dimension_semantics=("arbitrary",),
        ),
    )(hidden_states, weight)
